# baseline (device time: 14433 ns/iter reference)
import jax
import jax.numpy as jnp
from jax import lax
from jax.experimental import pallas as pl
from jax.experimental.pallas import tpu as pltpu

N_DEV = 4
N_EXP = 8
E_PER = 2
T_PER = 256
D_IN = 128
D_OUT = 256
CAPACITY = 102


def kernel(x, router_W, route_idx, expert_W):
    del router_W

    def body(x_ref, idx_ref, w_ref, out_ref,
             w16_out, comm_w, comm_i,
             send_w, recv_w, send_i, recv_i):
        my = lax.axis_index("i")

        barrier = pltpu.get_barrier_semaphore()
        for d in range(1, N_DEV):
            peer = lax.rem(my + d, N_DEV)
            pl.semaphore_signal(
                barrier, inc=1,
                device_id=(peer,), device_id_type=pl.DeviceIdType.MESH,
            )
        pl.semaphore_wait(barrier, N_DEV - 1)

        w16_out[:, :, :] = w_ref[:, :, :].astype(jnp.bfloat16)
        sends = []
        for d in range(1, N_DEV):
            peer = lax.rem(my + d, N_DEV)
            rw = pltpu.make_async_remote_copy(
                src_ref=w16_out, dst_ref=comm_w.at[my],
                send_sem=send_w.at[d - 1], recv_sem=recv_w.at[my],
                device_id=(peer,), device_id_type=pl.DeviceIdType.MESH,
            )
            ri = pltpu.make_async_remote_copy(
                src_ref=idx_ref, dst_ref=comm_i.at[my],
                send_sem=send_i.at[d - 1], recv_sem=recv_i.at[my],
                device_id=(peer,), device_id_type=pl.DeviceIdType.MESH,
            )
            rw.start()
            ri.start()
            sends.append((rw, ri))

        xv = x_ref[:, :]

        def gate_col(e_id):
            return (idx_ref[:, :] == e_id).astype(jnp.float32)

        acc = jnp.dot(xv * gate_col(my * E_PER), w_ref[0, :, :],
                      preferred_element_type=jnp.float32)
        acc = acc + jnp.dot(xv * gate_col(my * E_PER + 1), w_ref[1, :, :],
                            preferred_element_type=jnp.float32)

        exp_iota = lax.broadcasted_iota(jnp.int32, (T_PER, N_EXP), 1)
        onehot = (idx_ref[:, :] == exp_iota).astype(jnp.float32)
        row = lax.broadcasted_iota(jnp.int32, (T_PER, T_PER), 0)
        col = lax.broadcasted_iota(jnp.int32, (T_PER, T_PER), 1)
        tri = (col <= row).astype(jnp.float32)
        prefix = jnp.dot(tri, onehot, preferred_element_type=jnp.float32)

        base = jnp.zeros((1, N_EXP), jnp.float32)
        for d in range(1, N_DEV):
            peer = lax.rem(my + d, N_DEV)
            pltpu.make_async_remote_copy(
                src_ref=idx_ref, dst_ref=comm_i.at[peer],
                send_sem=send_i.at[d - 1], recv_sem=recv_i.at[peer],
                device_id=(peer,), device_id_type=pl.DeviceIdType.MESH,
            ).wait_recv()
            oh_p = (comm_i[peer, :, :] == exp_iota).astype(jnp.float32)
            cnt = jnp.sum(oh_p, axis=0, keepdims=True)
            base = base + jnp.where(peer < my, cnt,
                                    jnp.zeros((1, N_EXP), jnp.float32))

        rank = base + prefix
        rank_own = jnp.sum(rank * onehot, axis=1)
        keep = (rank_own <= float(CAPACITY)).astype(jnp.float32)

        for d in (1, 3, 2):
            peer = lax.rem(my + d, N_DEV)
            pltpu.make_async_remote_copy(
                src_ref=w16_out, dst_ref=comm_w.at[peer],
                send_sem=send_w.at[d - 1], recv_sem=recv_w.at[peer],
                device_id=(peer,), device_id_type=pl.DeviceIdType.MESH,
            ).wait_recv()
            wp = comm_w[peer, :, :, :].astype(jnp.float32)
            acc = acc + jnp.dot(xv * gate_col(peer * E_PER), wp[0, :, :],
                                preferred_element_type=jnp.float32)
            acc = acc + jnp.dot(xv * gate_col(peer * E_PER + 1), wp[1, :, :],
                                preferred_element_type=jnp.float32)

        out_ref[:, :] = keep[:, None] * acc

        for rw, ri in sends:
            rw.wait_send()
            ri.wait_send()

    return pl.pallas_call(
        body,
        out_shape=jax.ShapeDtypeStruct((T_PER, D_OUT), jnp.float32),
        in_specs=[
            pl.BlockSpec(memory_space=pltpu.VMEM),
            pl.BlockSpec(memory_space=pltpu.VMEM),
            pl.BlockSpec(memory_space=pltpu.VMEM),
        ],
        out_specs=pl.BlockSpec(memory_space=pltpu.VMEM),
        scratch_shapes=[
            pltpu.VMEM((E_PER, D_IN, D_OUT), jnp.bfloat16),
            pltpu.VMEM((N_DEV, E_PER, D_IN, D_OUT), jnp.bfloat16),
            pltpu.VMEM((N_DEV, T_PER, 1), jnp.int32),
            pltpu.SemaphoreType.DMA((N_DEV - 1,)),
            pltpu.SemaphoreType.DMA((N_DEV,)),
            pltpu.SemaphoreType.DMA((N_DEV - 1,)),
            pltpu.SemaphoreType.DMA((N_DEV,)),
        ],
        compiler_params=pltpu.CompilerParams(collective_id=0),
    )(x, route_idx, expert_W)
